# Initial kernel scaffold; baseline (speedup 1.0000x reference)
import functools

import jax
import jax.numpy as jnp
from jax import lax
from jax.experimental import pallas as pl
from jax.experimental.pallas import tpu as pltpu
from jax.experimental.pallas import tpu_sc as plsc

NC = 2
NS = 16
NW = NC * NS
LANES = 16


def _sc_degree(dst, zeros_hist, n_pad):
    ep = dst.shape[0]
    e_per_tile = ep // NW
    mesh = plsc.VectorSubcoreMesh(core_axis_name="c", subcore_axis_name="s")

    @functools.partial(
        pl.kernel,
        out_type=jax.ShapeDtypeStruct((NW, n_pad), jnp.float32),
        mesh=mesh,
        scratch_types=[
            pltpu.VMEM((e_per_tile,), jnp.int32),
            pltpu.VMEM((n_pad,), jnp.float32),
        ],
    )
    def deg_kernel(dst_hbm, zeros_hbm, out_hbm, dst_v, hist):
        c = lax.axis_index("c")
        s = lax.axis_index("s")
        wid = c * NS + s
        pltpu.sync_copy(dst_hbm.at[pl.ds(wid * e_per_tile, e_per_tile)], dst_v)
        pltpu.sync_copy(zeros_hbm, hist)
        ones = jnp.ones((LANES,), jnp.float32)

        @pl.loop(0, e_per_tile // LANES)
        def _(i):
            d = dst_v[pl.ds(i * LANES, LANES)]
            plsc.addupdate_scatter(hist, [d], ones)

        pltpu.sync_copy(hist, out_hbm.at[wid])

    return deg_kernel(dst, zeros_hist)


def _tc_transform(x_pad, W, deg_parts):
    n_pad, din = x_pad.shape
    dout = W.shape[1]
    blk = 1280
    grid = n_pad // blk

    def body(x_ref, w_ref, deg_ref, h2_ref, dinv_ref):
        deg = jnp.sum(deg_ref[...], axis=0) + 1.0
        dinv = lax.rsqrt(deg)
        h = jnp.dot(x_ref[...], w_ref[...], preferred_element_type=jnp.float32)
        h2_ref[...] = h * dinv[:, None]
        dinv_ref[...] = dinv

    return pl.pallas_call(
        body,
        grid=(grid,),
        in_specs=[
            pl.BlockSpec((blk, din), lambda i: (i, 0)),
            pl.BlockSpec((din, dout), lambda i: (0, 0)),
            pl.BlockSpec((NW, blk), lambda i: (0, i)),
        ],
        out_specs=[
            pl.BlockSpec((blk, dout), lambda i: (i, 0)),
            pl.BlockSpec((blk,), lambda i: (i,)),
        ],
        out_shape=[
            jax.ShapeDtypeStruct((n_pad, dout), jnp.float32),
            jax.ShapeDtypeStruct((n_pad,), jnp.float32),
        ],
    )(x_pad, W, deg_parts)


def _sc_messages(src_t, dst_t, h2, zeros_tile):
    _, ch, ck = src_t.shape
    n_pad, dout = h2.shape
    rows_per_tile = n_pad // NS
    n_zero_copies = rows_per_tile // ck
    mesh = plsc.VectorSubcoreMesh(core_axis_name="c", subcore_axis_name="s")

    @functools.partial(
        pl.kernel,
        out_type=jax.ShapeDtypeStruct((NC, n_pad, dout), jnp.float32),
        mesh=mesh,
        scratch_types=[
            pltpu.VMEM((ch, ck), jnp.int32),
            pltpu.VMEM((ch, ck), jnp.int32),
            pltpu.VMEM((ck, dout), jnp.float32),
            pltpu.VMEM_SHARED((n_pad, dout), jnp.float32),
        ],
    )
    def msg_kernel(src_hbm, dst_hbm, h2_hbm, z_hbm, out_hbm,
                   src_v, dst_v, rowbuf, acc):
        c = lax.axis_index("c")
        s = lax.axis_index("s")
        wid = c * NS + s
        pltpu.sync_copy(src_hbm.at[wid], src_v)
        pltpu.sync_copy(dst_hbm.at[wid], dst_v)
        pltpu.sync_copy(z_hbm, rowbuf)
        for k in range(n_zero_copies):
            pltpu.sync_copy(rowbuf, acc.at[pl.ds(s * rows_per_tile + k * ck, ck)])
        plsc.subcore_barrier()

        @pl.loop(0, ch)
        def _(j):
            pltpu.sync_copy(h2_hbm.at[src_v.at[j]], rowbuf)
            pltpu.sync_copy(rowbuf, acc.at[dst_v.at[j]], add=True)

        plsc.subcore_barrier()
        pltpu.sync_copy(acc.at[pl.ds(s * rows_per_tile, rows_per_tile)],
                        out_hbm.at[c, pl.ds(s * rows_per_tile, rows_per_tile)])

    return msg_kernel(src_t, dst_t, h2, zeros_tile)


def _tc_epilogue(parts, h2, dinv, b, n_out):
    n_pad, dout = h2.shape
    blk = 2000
    grid = n_out // blk

    def body(p_ref, h2_ref, dinv_ref, b_ref, o_ref):
        tot = p_ref[0] + p_ref[1] + h2_ref[...]
        o_ref[...] = jnp.maximum(
            tot * dinv_ref[...][:, None] + b_ref[...][None, :], 0.0)

    return pl.pallas_call(
        body,
        grid=(grid,),
        in_specs=[
            pl.BlockSpec((NC, blk, dout), lambda i: (0, i, 0)),
            pl.BlockSpec((blk, dout), lambda i: (i, 0)),
            pl.BlockSpec((blk,), lambda i: (i,)),
            pl.BlockSpec((dout,), lambda i: (0,)),
        ],
        out_specs=pl.BlockSpec((blk, dout), lambda i: (i, 0)),
        out_shape=jax.ShapeDtypeStruct((n_out, dout), jnp.float32),
    )(parts, h2, dinv, b)


def kernel(x, edge_index, W, b):
    n, din = x.shape
    dout = W.shape[1]
    e = edge_index.shape[1]
    src = edge_index[0].astype(jnp.int32)
    dst = edge_index[1].astype(jnp.int32)

    ck = 128
    n_pad = ((n + NS * ck - 1) // (NS * ck)) * (NS * ck)
    x_pad = jnp.concatenate(
        [x, jnp.zeros((n_pad - n, din), jnp.float32)], axis=0)

    e_deg = ((e + NW * LANES - 1) // (NW * LANES)) * (NW * LANES)
    dst_deg = jnp.concatenate(
        [dst, jnp.full((e_deg - e,), n, jnp.int32)])
    zeros_hist = jnp.zeros((n_pad,), jnp.float32)
    deg_parts = _sc_degree(dst_deg, zeros_hist, n_pad)

    h2, dinv = _tc_transform(x_pad, W, deg_parts)

    e_pad = ((e + NW * ck - 1) // (NW * ck)) * (NW * ck)
    pad = jnp.full((e_pad - e,), n, jnp.int32)
    src_t = jnp.concatenate([src, pad]).reshape(NW, e_pad // (NW * ck), ck)
    dst_t = jnp.concatenate([dst, pad]).reshape(NW, e_pad // (NW * ck), ck)
    zeros_tile = jnp.zeros((ck, dout), jnp.float32)
    parts = _sc_messages(src_t, dst_t, h2, zeros_tile)

    return _tc_epilogue(parts, h2, dinv, b, n)


# speedup vs baseline: 20.8191x; 20.8191x over previous
import dataclasses
import functools

import jax
import jax.numpy as jnp
from jax import lax
from jax.experimental import pallas as pl
from jax.experimental.pallas import tpu as pltpu
from jax.experimental.pallas import tpu_sc as plsc

NC = 2
NS = 16
NW = NC * NS
LANES = 16


def _sc_compiler_params():
    cp = pltpu.CompilerParams()
    if "needs_layout_passes" in pltpu.CompilerParams.__dataclass_fields__:
        cp = dataclasses.replace(cp, needs_layout_passes=False)
    return cp


def _sc_degree(dst, zeros_hist, n_pad):
    ep = dst.shape[0]
    e_per_tile = ep // NW
    mesh = plsc.VectorSubcoreMesh(core_axis_name="c", subcore_axis_name="s")

    @functools.partial(
        pl.kernel,
        out_type=jax.ShapeDtypeStruct((NW, n_pad), jnp.float32),
        mesh=mesh,
        scratch_types=[
            pltpu.VMEM((e_per_tile,), jnp.int32),
            pltpu.VMEM((n_pad,), jnp.float32),
        ],
        compiler_params=_sc_compiler_params(),
    )
    def deg_kernel(dst_hbm, zeros_hbm, out_hbm, dst_v, hist):
        c = lax.axis_index("c")
        s = lax.axis_index("s")
        wid = c * NS + s
        pltpu.sync_copy(dst_hbm.at[pl.ds(wid * e_per_tile, e_per_tile)], dst_v)
        pltpu.sync_copy(zeros_hbm, hist)
        ones = jnp.ones((LANES,), jnp.float32)

        @pl.loop(0, e_per_tile // LANES)
        def _(i):
            d = dst_v[pl.ds(i * LANES, LANES)]
            plsc.addupdate_scatter(hist, [d], ones)

        pltpu.sync_copy(hist, out_hbm.at[wid])

    return deg_kernel(dst, zeros_hist)


def _tc_transform(x_pad, W, deg_parts):
    n_pad, din = x_pad.shape
    dout = W.shape[1]
    blk = 1280
    grid = n_pad // blk

    def body(x_ref, w_ref, deg_ref, h2_ref, dinv_ref):
        deg = jnp.sum(deg_ref[...], axis=0) + 1.0
        dinv = lax.rsqrt(deg)
        h = jnp.dot(x_ref[...], w_ref[...], preferred_element_type=jnp.float32)
        h2_ref[...] = h * dinv[:, None]
        dinv_ref[pl.ds(pl.program_id(0) * blk, blk)] = dinv

    return pl.pallas_call(
        body,
        grid=(grid,),
        in_specs=[
            pl.BlockSpec((blk, din), lambda i: (i, 0)),
            pl.BlockSpec((din, dout), lambda i: (0, 0)),
            pl.BlockSpec((NW, blk), lambda i: (0, i)),
        ],
        out_specs=[
            pl.BlockSpec((blk, dout), lambda i: (i, 0)),
            pl.BlockSpec((n_pad,), lambda i: (0,)),
        ],
        out_shape=[
            jax.ShapeDtypeStruct((n_pad, dout), jnp.float32),
            jax.ShapeDtypeStruct((n_pad,), jnp.float32),
        ],
    )(x_pad, W, deg_parts)


def _sc_messages(src_t, dst_t, h2, zeros_tile):
    _, ch, ck = src_t.shape
    n_pad, dout = h2.shape
    rows_per_tile = n_pad // NS
    n_zero_copies = rows_per_tile // ck
    mesh = plsc.VectorSubcoreMesh(core_axis_name="c", subcore_axis_name="s")

    @functools.partial(
        pl.kernel,
        out_type=jax.ShapeDtypeStruct((NC, n_pad, dout), jnp.float32),
        mesh=mesh,
        scratch_types=[
            pltpu.VMEM((ch, ck), jnp.int32),
            pltpu.VMEM((ch, ck), jnp.int32),
            pltpu.VMEM((ck, dout), jnp.float32),
            pltpu.VMEM_SHARED((n_pad, dout), jnp.float32),
        ],
        compiler_params=_sc_compiler_params(),
    )
    def msg_kernel(src_hbm, dst_hbm, h2_hbm, z_hbm, out_hbm,
                   src_v, dst_v, rowbuf, acc):
        c = lax.axis_index("c")
        s = lax.axis_index("s")
        wid = c * NS + s
        pltpu.sync_copy(src_hbm.at[wid], src_v)
        pltpu.sync_copy(dst_hbm.at[wid], dst_v)
        pltpu.sync_copy(z_hbm, rowbuf)
        for k in range(n_zero_copies):
            pltpu.sync_copy(rowbuf, acc.at[pl.ds(s * rows_per_tile + k * ck, ck)])
        plsc.subcore_barrier()

        @pl.loop(0, ch)
        def _(j):
            pltpu.sync_copy(h2_hbm.at[src_v.at[j]], rowbuf)
            pltpu.sync_copy(rowbuf, acc.at[dst_v.at[j]], add=True)

        plsc.subcore_barrier()
        pltpu.sync_copy(acc.at[pl.ds(s * rows_per_tile, rows_per_tile)],
                        out_hbm.at[c, pl.ds(s * rows_per_tile, rows_per_tile)])

    return msg_kernel(src_t, dst_t, h2, zeros_tile)


def _tc_epilogue(parts, h2, dinv, b):
    n_pad, dout = h2.shape
    blk = 2048
    grid = n_pad // blk

    def body(p_ref, h2_ref, dinv_ref, b_ref, o_ref):
        tot = p_ref[0] + p_ref[1] + h2_ref[...]
        dinv = dinv_ref[pl.ds(pl.program_id(0) * blk, blk)]
        o_ref[...] = jnp.maximum(
            tot * dinv[:, None] + b_ref[...][None, :], 0.0)

    return pl.pallas_call(
        body,
        grid=(grid,),
        in_specs=[
            pl.BlockSpec((NC, blk, dout), lambda i: (0, i, 0)),
            pl.BlockSpec((blk, dout), lambda i: (i, 0)),
            pl.BlockSpec((n_pad,), lambda i: (0,)),
            pl.BlockSpec((dout,), lambda i: (0,)),
        ],
        out_specs=pl.BlockSpec((blk, dout), lambda i: (i, 0)),
        out_shape=jax.ShapeDtypeStruct((n_pad, dout), jnp.float32),
    )(parts, h2, dinv, b)


def kernel(x, edge_index, W, b):
    n, din = x.shape
    dout = W.shape[1]
    e = edge_index.shape[1]
    src = edge_index[0].astype(jnp.int32)
    dst = edge_index[1].astype(jnp.int32)

    ck = 128
    n_pad = ((n + NS * ck - 1) // (NS * ck)) * (NS * ck)
    x_pad = jnp.concatenate(
        [x, jnp.zeros((n_pad - n, din), jnp.float32)], axis=0)

    e_deg = ((e + NW * LANES - 1) // (NW * LANES)) * (NW * LANES)
    dst_deg = jnp.concatenate(
        [dst, jnp.full((e_deg - e,), n, jnp.int32)])
    zeros_hist = jnp.zeros((n_pad,), jnp.float32)
    deg_parts = _sc_degree(dst_deg, zeros_hist, n_pad)

    h2, dinv = _tc_transform(x_pad, W, deg_parts)

    e_pad = ((e + NW * ck - 1) // (NW * ck)) * (NW * ck)
    pad = jnp.full((e_pad - e,), n, jnp.int32)
    src_t = jnp.concatenate([src, pad]).reshape(NW, e_pad // (NW * ck), ck)
    dst_t = jnp.concatenate([dst, pad]).reshape(NW, e_pad // (NW * ck), ck)
    zeros_tile = jnp.zeros((ck, dout), jnp.float32)
    parts = _sc_messages(src_t, dst_t, h2, zeros_tile)

    return _tc_epilogue(parts, h2, dinv, b)[:n]
